# Initial kernel scaffold; baseline (speedup 1.0000x reference)
import functools

import jax
import jax.numpy as jnp
from jax import lax
from jax.experimental import pallas as pl
from jax.experimental.pallas import tpu as pltpu
from jax.experimental.pallas import tpu_sc as plsc

K = 100
STATE_DIM = 64
NUM_ACTIONS = 100000
BATCH = 1024

TILE = 512
GRID = 196
NPAD = GRID * TILE
NC1 = NPAD // 16
NC2 = NC1 // 16
NC2P = 400
NC3 = 25
OUTW = 112
NWORK = 32
RPW = BATCH // NWORK
NEG = -1.0


def _a0_body(state_ref, w_ref, b_ref, m_ref, ms):
    j = pl.program_id(0)
    s = jnp.dot(state_ref[...], w_ref[...],
                preferred_element_type=jnp.float32) + b_ref[...]
    tm = jnp.max(s, axis=1, keepdims=True)

    @pl.when(j == 0)
    def _():
        ms[...] = tm

    @pl.when(j > 0)
    def _():
        ms[...] = jnp.maximum(ms[...], tm)

    @pl.when(j == GRID - 1)
    def _():
        m_ref[...] = ms[...]


def _a1_body(state_ref, w_ref, b_ref, m_ref, u_ref, l1_ref, l2_ref, z_ref, zs):
    j = pl.program_id(0)
    s = jnp.dot(state_ref[...], w_ref[...],
                preferred_element_type=jnp.float32) + b_ref[...]
    u = jnp.exp(s - m_ref[...])
    u_ref[...] = u
    l1 = jnp.max(u.reshape(BATCH, TILE // 16, 16), axis=2)
    l1_ref[...] = l1
    l2_ref[...] = jnp.max(l1.reshape(BATCH, TILE // 256, 16), axis=2)
    ts = jnp.sum(u, axis=1, keepdims=True)

    @pl.when(j == 0)
    def _():
        zs[...] = ts

    @pl.when(j > 0)
    def _():
        zs[...] = zs[...] + ts

    @pl.when(j == GRID - 1)
    def _():
        z_ref[...] = zs[...]


def _scalar(x):
    if getattr(x, "ndim", 0) == 0:
        return x
    return lax.reduce_max(x, axes=tuple(range(x.ndim)))


def _sc_body(u_hbm, l1_hbm, l2_hbm, z_hbm, uo_hbm, io_hbm,
             datav, l1v, l2v, l3v, uov, iov, zv):
    iota16 = lax.iota(jnp.int32, 16)
    lane0 = iota16 == 0
    wid = lax.axis_index("s") * 2 + lax.axis_index("c")
    row0 = wid * RPW
    pltpu.sync_copy(z_hbm.at[pl.ds(row0, RPW)], zv)

    def row_body(t, _):
        row = row0 + t
        pltpu.sync_copy(u_hbm.at[row], datav)
        pltpu.sync_copy(l1_hbm.at[row], l1v)
        pltpu.sync_copy(l2_hbm.at[row], l2v)

        l3v[pl.ds(0, 16)] = jnp.full((16,), NEG, jnp.float32)
        l3v[pl.ds(16, 16)] = jnp.full((16,), NEG, jnp.float32)

        def build3(i, _):
            nm = _scalar(lax.reduce_max(
                l2v[pl.ds(pl.multiple_of(i * 16, 16), 16)], axes=(0,)))
            base = (i // 16) * 16
            lane = i - base
            v = l3v[pl.ds(pl.multiple_of(base, 16), 16)]
            l3v[pl.ds(pl.multiple_of(base, 16), 16)] = jnp.where(
                iota16 == lane, nm, v)
            return 0

        lax.fori_loop(0, NC3, build3, 0)

        def ext_body(e, _):
            v3a = l3v[pl.ds(0, 16)]
            v3b = l3v[pl.ds(16, 16)]
            bsel = v3b > v3a
            accv = jnp.where(bsel, v3b, v3a)
            accb = jnp.where(bsel, 1, 0)
            m = _scalar(lax.reduce_max(accv, axes=(0,)))
            cand = jnp.where(accv == m, iota16 + accb * 16, 1 << 30)
            j3 = _scalar(lax.reduce_min(cand, axes=(0,)))
            v2 = l2v[pl.ds(pl.multiple_of(j3 * 16, 16), 16)]
            l2l = _scalar(plsc.all_reduce_ffs(v2 == m))
            j2 = j3 * 16 + l2l
            v1 = l1v[pl.ds(pl.multiple_of(j2 * 16, 16), 16)]
            l1l = _scalar(plsc.all_reduce_ffs(v1 == m))
            j1 = j2 * 16 + l1l
            v0 = datav[pl.ds(pl.multiple_of(j1 * 16, 16), 16)]
            l0l = _scalar(plsc.all_reduce_ffs(v0 == m))
            g = j1 * 16 + l0l
            plsc.store_scatter(uov, [jnp.full((16,), e, jnp.int32)],
                               jnp.full((16,), m, jnp.float32), mask=lane0)
            plsc.store_scatter(iov, [jnp.full((16,), e, jnp.int32)],
                               jnp.full((16,), g, jnp.int32), mask=lane0)
            v0n = jnp.where(iota16 == l0l, jnp.float32(NEG), v0)
            datav[pl.ds(pl.multiple_of(j1 * 16, 16), 16)] = v0n
            n1 = _scalar(lax.reduce_max(v0n, axes=(0,)))
            v1n = jnp.where(iota16 == l1l, n1, v1)
            l1v[pl.ds(pl.multiple_of(j2 * 16, 16), 16)] = v1n
            n2 = _scalar(lax.reduce_max(v1n, axes=(0,)))
            v2n = jnp.where(iota16 == l2l, n2, v2)
            l2v[pl.ds(pl.multiple_of(j3 * 16, 16), 16)] = v2n
            n3 = _scalar(lax.reduce_max(v2n, axes=(0,)))
            b3 = (j3 // 16) * 16
            l3l = j3 - b3
            v3 = l3v[pl.ds(pl.multiple_of(b3, 16), 16)]
            l3v[pl.ds(pl.multiple_of(b3, 16), 16)] = jnp.where(
                iota16 == l3l, n3, v3)
            return 0

        lax.fori_loop(0, K, ext_body, 0)

        tb = (t // 16) * 16
        vz = zv[pl.ds(pl.multiple_of(tb, 16), 16)]
        z = _scalar(lax.reduce_max(
            jnp.where(iota16 == (t - tb), vz, jnp.float32(0.0)), axes=(0,)))
        for q in range(OUTW // 16):
            uov[pl.ds(q * 16, 16)] = uov[pl.ds(q * 16, 16)] / z

        pltpu.sync_copy(uov, uo_hbm.at[row])
        pltpu.sync_copy(iov, io_hbm.at[row])
        return 0

    lax.fori_loop(0, RPW, row_body, 0)


@jax.jit
def kernel(state, W, b):
    f32 = jnp.float32
    wp = jnp.concatenate(
        [W, jnp.zeros((STATE_DIM, NPAD - NUM_ACTIONS), f32)], axis=1)
    bp = jnp.concatenate(
        [b, jnp.full((NPAD - NUM_ACTIONS,), -jnp.inf, f32)]).reshape(1, NPAD)

    m = pl.pallas_call(
        _a0_body,
        grid=(GRID,),
        in_specs=[
            pl.BlockSpec((BATCH, STATE_DIM), lambda j: (0, 0)),
            pl.BlockSpec((STATE_DIM, TILE), lambda j: (0, j)),
            pl.BlockSpec((1, TILE), lambda j: (0, j)),
        ],
        out_specs=pl.BlockSpec((BATCH, 1), lambda j: (0, 0)),
        out_shape=jax.ShapeDtypeStruct((BATCH, 1), f32),
        scratch_shapes=[pltpu.VMEM((BATCH, 1), f32)],
    )(state, wp, bp)

    u, l1, l2, z = pl.pallas_call(
        _a1_body,
        grid=(GRID,),
        in_specs=[
            pl.BlockSpec((BATCH, STATE_DIM), lambda j: (0, 0)),
            pl.BlockSpec((STATE_DIM, TILE), lambda j: (0, j)),
            pl.BlockSpec((1, TILE), lambda j: (0, j)),
            pl.BlockSpec((BATCH, 1), lambda j: (0, 0)),
        ],
        out_specs=[
            pl.BlockSpec((BATCH, TILE), lambda j: (0, j)),
            pl.BlockSpec((BATCH, TILE // 16), lambda j: (0, j)),
            pl.BlockSpec((BATCH, TILE // 256), lambda j: (0, j)),
            pl.BlockSpec((BATCH, 1), lambda j: (0, 0)),
        ],
        out_shape=[
            jax.ShapeDtypeStruct((BATCH, NPAD), f32),
            jax.ShapeDtypeStruct((BATCH, NC1), f32),
            jax.ShapeDtypeStruct((BATCH, NC2), f32),
            jax.ShapeDtypeStruct((BATCH, 1), f32),
        ],
        scratch_shapes=[pltpu.VMEM((BATCH, 1), f32)],
    )(state, wp, bp, m)

    l2p = jnp.concatenate(
        [l2, jnp.full((BATCH, NC2P - NC2), NEG, f32)], axis=1)
    zflat = z.reshape(BATCH)

    sc = functools.partial(
        pl.kernel,
        out_type=[
            jax.ShapeDtypeStruct((BATCH, OUTW), f32),
            jax.ShapeDtypeStruct((BATCH, OUTW), jnp.int32),
        ],
        mesh=plsc.VectorSubcoreMesh(core_axis_name="c", subcore_axis_name="s"),
        scratch_types=[
            pltpu.VMEM((NPAD,), f32),
            pltpu.VMEM((NC1,), f32),
            pltpu.VMEM((NC2P,), f32),
            pltpu.VMEM((32,), f32),
            pltpu.VMEM((OUTW,), f32),
            pltpu.VMEM((OUTW,), jnp.int32),
            pltpu.VMEM((RPW,), f32),
        ],
    )(_sc_body)

    uo, io = sc(u, l1, l2p, zflat)
    return (io[:, :K], uo[:, :K])


# speedup vs baseline: 24.9713x; 24.9713x over previous
import functools

import jax
import jax.numpy as jnp
from jax import lax
from jax.experimental import pallas as pl
from jax.experimental.pallas import tpu as pltpu
from jax.experimental.pallas import tpu_sc as plsc

K = 100
STATE_DIM = 64
NUM_ACTIONS = 100000
BATCH = 1024

AT = 2048
GA = 49
RB = 256
GR = BATCH // RB
NPAD = GA * AT
NC1 = NPAD // 16
NC2 = NC1 // 16
NC2P = 400
NC3 = 25
OUTW = 112
NWORK = 32
RPW = BATCH // NWORK
NEG = -1.0


def _chunkmax16(x, rows, cols):
    return jnp.max(x.reshape(rows, cols // 16, 16), axis=2)


def _a0_body(state_ref, w_ref, b_ref, m_ref, ms):
    j = pl.program_id(1)
    s = jnp.dot(state_ref[...], w_ref[...],
                preferred_element_type=jnp.float32) + b_ref[...]
    tm = jnp.max(s, axis=1, keepdims=True)

    @pl.when(j == 0)
    def _():
        ms[...] = tm

    @pl.when(j > 0)
    def _():
        ms[...] = jnp.maximum(ms[...], tm)

    @pl.when(j == GA - 1)
    def _():
        m_ref[...] = ms[...]


def _a1_body(state_ref, w_ref, b_ref, m_ref, u_ref, l1_ref, z_ref, zs):
    j = pl.program_id(1)
    s = jnp.dot(state_ref[...], w_ref[...],
                preferred_element_type=jnp.float32) + b_ref[...]
    u = jnp.exp(s - m_ref[...])
    u_ref[...] = u
    l1_ref[...] = _chunkmax16(u, RB, AT)
    ts = jnp.sum(u, axis=1, keepdims=True)

    @pl.when(j == 0)
    def _():
        zs[...] = ts

    @pl.when(j > 0)
    def _():
        zs[...] = zs[...] + ts

    @pl.when(j == GA - 1)
    def _():
        z_ref[...] = zs[...]


RB2 = 32


def _a2_body(l1_ref, l2_ref):
    l2_ref[...] = _chunkmax16(l1_ref[...], RB2, NC1)


def _iota16():
    return lax.iota(jnp.int32, 16)


def _gather16(v, idx):
    dn = lax.GatherDimensionNumbers(
        offset_dims=(), collapsed_slice_dims=(0,), start_index_map=(0,))
    return lax.gather(v, idx[:, None], dn, (1,),
                      mode=lax.GatherScatterMode.PROMISE_IN_BOUNDS)


def _bf_max(v):
    for sh in (8, 4, 2, 1):
        v = jnp.maximum(v, _gather16(v, _iota16() ^ sh))
    return v


def _bf_min_i32(v):
    for sh in (8, 4, 2, 1):
        v = jnp.minimum(v, _gather16(v, _iota16() ^ sh))
    return v


def _first_set(mask):
    return _bf_min_i32(jnp.where(mask, _iota16(), 16))[0]


def _sc_body(u_hbm, l1_hbm, l2_hbm, z_hbm, uo_hbm, io_hbm,
             datav, l1v, l2v, l3v, uov, iov, zv):
    iota = _iota16()
    wid = lax.axis_index("s") * 2 + lax.axis_index("c")
    row0 = wid * RPW
    pltpu.sync_copy(z_hbm.at[pl.ds(pl.multiple_of(row0, 8), RPW)], zv)

    def row_body(t, _):
        row = row0 + t
        pltpu.sync_copy(u_hbm.at[pl.ds(pl.multiple_of(row * NPAD, 8), NPAD)],
                        datav)
        pltpu.sync_copy(l1_hbm.at[pl.ds(pl.multiple_of(row * NC1, 8), NC1)],
                        l1v)
        pltpu.sync_copy(l2_hbm.at[pl.ds(pl.multiple_of(row * NC2P, 8), NC2P)],
                        l2v)

        l3v[pl.ds(0, 16)] = jnp.full((16,), NEG, jnp.float32)
        l3v[pl.ds(16, 16)] = jnp.full((16,), NEG, jnp.float32)

        def build3(i, _):
            nm = _bf_max(l2v[pl.ds(pl.multiple_of(i * 16, 16), 16)])
            base = (i // 16) * 16
            lane = i - base
            v = l3v[pl.ds(pl.multiple_of(base, 16), 16)]
            l3v[pl.ds(pl.multiple_of(base, 16), 16)] = jnp.where(
                iota == lane, nm, v)
            return 0

        lax.fori_loop(0, NC3, build3, 0)

        def ext_body(e, _):
            v3a = l3v[pl.ds(0, 16)]
            v3b = l3v[pl.ds(16, 16)]
            bsel = v3b > v3a
            accv = jnp.where(bsel, v3b, v3a)
            accb = jnp.where(bsel, 16, 0)
            mv = _bf_max(accv)
            cand = jnp.where(accv == mv, iota + accb, 1 << 30)
            j3 = _bf_min_i32(cand)[0]
            v2 = l2v[pl.ds(pl.multiple_of(j3 * 16, 16), 16)]
            l2l = _first_set(v2 == mv)
            j2 = j3 * 16 + l2l
            v1 = l1v[pl.ds(pl.multiple_of(j2 * 16, 16), 16)]
            l1l = _first_set(v1 == mv)
            j1 = j2 * 16 + l1l
            v0 = datav[pl.ds(pl.multiple_of(j1 * 16, 16), 16)]
            l0l = _first_set(v0 == mv)
            g = j1 * 16 + l0l
            ob = (e // 16) * 16
            ol = e - ob
            uo = uov[pl.ds(pl.multiple_of(ob, 16), 16)]
            uov[pl.ds(pl.multiple_of(ob, 16), 16)] = jnp.where(
                iota == ol, mv, uo)
            io = iov[pl.ds(pl.multiple_of(ob, 16), 16)]
            iov[pl.ds(pl.multiple_of(ob, 16), 16)] = jnp.where(
                iota == ol, g, io)
            v0n = jnp.where(iota == l0l, jnp.float32(NEG), v0)
            datav[pl.ds(pl.multiple_of(j1 * 16, 16), 16)] = v0n
            n1 = _bf_max(v0n)
            v1n = jnp.where(iota == l1l, n1, v1)
            l1v[pl.ds(pl.multiple_of(j2 * 16, 16), 16)] = v1n
            n2 = _bf_max(v1n)
            v2n = jnp.where(iota == l2l, n2, v2)
            l2v[pl.ds(pl.multiple_of(j3 * 16, 16), 16)] = v2n
            n3 = _bf_max(v2n)
            b3 = (j3 // 16) * 16
            l3l = j3 - b3
            v3 = l3v[pl.ds(pl.multiple_of(b3, 16), 16)]
            l3v[pl.ds(pl.multiple_of(b3, 16), 16)] = jnp.where(
                iota == l3l, n3, v3)
            return 0

        lax.fori_loop(0, K, ext_body, 0)

        tb = (t // 16) * 16
        vz = zv[pl.ds(pl.multiple_of(tb, 16), 16)]
        zvec = _bf_max(jnp.where(iota == (t - tb), vz, jnp.float32(0.0)))
        for q in range(OUTW // 16):
            uov[pl.ds(q * 16, 16)] = uov[pl.ds(q * 16, 16)] / zvec

        pltpu.sync_copy(uov, uo_hbm.at[pl.ds(pl.multiple_of(row * OUTW, 8),
                                             OUTW)])
        pltpu.sync_copy(iov, io_hbm.at[pl.ds(pl.multiple_of(row * OUTW, 8),
                                             OUTW)])
        return 0

    lax.fori_loop(0, RPW, row_body, 0)


@jax.jit
def kernel(state, W, b):
    f32 = jnp.float32
    wp = jnp.concatenate(
        [W, jnp.zeros((STATE_DIM, NPAD - NUM_ACTIONS), f32)], axis=1)
    bp = jnp.concatenate(
        [b, jnp.full((NPAD - NUM_ACTIONS,), -jnp.inf, f32)]).reshape(1, NPAD)

    m = pl.pallas_call(
        _a0_body,
        grid=(GR, GA),
        in_specs=[
            pl.BlockSpec((RB, STATE_DIM), lambda i, j: (i, 0)),
            pl.BlockSpec((STATE_DIM, AT), lambda i, j: (0, j)),
            pl.BlockSpec((1, AT), lambda i, j: (0, j)),
        ],
        out_specs=pl.BlockSpec((RB, 1), lambda i, j: (i, 0)),
        out_shape=jax.ShapeDtypeStruct((BATCH, 1), f32),
        scratch_shapes=[pltpu.VMEM((RB, 1), f32)],
    )(state, wp, bp)

    u, l1, z = pl.pallas_call(
        _a1_body,
        grid=(GR, GA),
        in_specs=[
            pl.BlockSpec((RB, STATE_DIM), lambda i, j: (i, 0)),
            pl.BlockSpec((STATE_DIM, AT), lambda i, j: (0, j)),
            pl.BlockSpec((1, AT), lambda i, j: (0, j)),
            pl.BlockSpec((RB, 1), lambda i, j: (i, 0)),
        ],
        out_specs=[
            pl.BlockSpec((RB, AT), lambda i, j: (i, j)),
            pl.BlockSpec((RB, AT // 16), lambda i, j: (i, j)),
            pl.BlockSpec((RB, 1), lambda i, j: (i, 0)),
        ],
        out_shape=[
            jax.ShapeDtypeStruct((BATCH, NPAD), f32),
            jax.ShapeDtypeStruct((BATCH, NC1), f32),
            jax.ShapeDtypeStruct((BATCH, 1), f32),
        ],
        scratch_shapes=[pltpu.VMEM((RB, 1), f32)],
    )(state, wp, bp, m)

    l2 = pl.pallas_call(
        _a2_body,
        grid=(BATCH // RB2,),
        in_specs=[pl.BlockSpec((RB2, NC1), lambda i: (i, 0))],
        out_specs=pl.BlockSpec((RB2, NC2), lambda i: (i, 0)),
        out_shape=jax.ShapeDtypeStruct((BATCH, NC2), f32),
    )(l1)

    l2p = jnp.concatenate(
        [l2, jnp.full((BATCH, NC2P - NC2), NEG, f32)], axis=1)

    sc = functools.partial(
        pl.kernel,
        out_type=[
            jax.ShapeDtypeStruct((BATCH * OUTW,), f32),
            jax.ShapeDtypeStruct((BATCH * OUTW,), jnp.int32),
        ],
        mesh=plsc.VectorSubcoreMesh(core_axis_name="c", subcore_axis_name="s"),
        scratch_types=[
            pltpu.VMEM((NPAD,), f32),
            pltpu.VMEM((NC1,), f32),
            pltpu.VMEM((NC2P,), f32),
            pltpu.VMEM((32,), f32),
            pltpu.VMEM((OUTW,), f32),
            pltpu.VMEM((OUTW,), jnp.int32),
            pltpu.VMEM((RPW,), f32),
        ],
    )(_sc_body)

    uo, io = sc(u.reshape(BATCH * NPAD), l1.reshape(BATCH * NC1),
                l2p.reshape(BATCH * NC2P), z.reshape(BATCH))
    uo = uo.reshape(BATCH, OUTW)
    io = io.reshape(BATCH, OUTW)
    return (io[:, :K], uo[:, :K])


# speedup vs baseline: 26.9441x; 1.0790x over previous
import functools

import jax
import jax.numpy as jnp
from jax import lax
from jax.experimental import pallas as pl
from jax.experimental.pallas import tpu as pltpu
from jax.experimental.pallas import tpu_sc as plsc

K = 100
STATE_DIM = 64
NUM_ACTIONS = 100000
BATCH = 1024

AT = 2048
GA = 49
RB = 256
GR = BATCH // RB
NPAD = GA * AT
NC1 = NPAD // 16
NC2 = NC1 // 16
NC2P = 400
NC3 = 25
OUTW = 112
NWORK = 32
RPW = BATCH // NWORK
NEG = -1.0


def _chunkmax16(x, rows, cols):
    return jnp.max(x.reshape(rows, cols // 16, 16), axis=2)


def _a0_body(state_ref, w_ref, b_ref, m_ref, ms):
    j = pl.program_id(1)
    s = jnp.dot(state_ref[...], w_ref[...],
                preferred_element_type=jnp.float32) + b_ref[...]
    tm = jnp.max(s, axis=1, keepdims=True)

    @pl.when(j == 0)
    def _():
        ms[...] = tm

    @pl.when(j > 0)
    def _():
        ms[...] = jnp.maximum(ms[...], tm)

    @pl.when(j == GA - 1)
    def _():
        m_ref[...] = ms[...]


def _a1_body(state_ref, w_ref, b_ref, m_ref, u_ref, l1_ref, z_ref, zs):
    j = pl.program_id(1)
    s = jnp.dot(state_ref[...], w_ref[...],
                preferred_element_type=jnp.float32) + b_ref[...]
    u = jnp.exp(s - m_ref[...])
    u_ref[...] = u
    l1_ref[...] = _chunkmax16(u, RB, AT)
    ts = jnp.sum(u, axis=1, keepdims=True)

    @pl.when(j == 0)
    def _():
        zs[...] = ts

    @pl.when(j > 0)
    def _():
        zs[...] = zs[...] + ts

    @pl.when(j == GA - 1)
    def _():
        z_ref[...] = zs[...]


RB2 = 32


def _a2_body(l1_ref, l2_ref):
    l2_ref[...] = _chunkmax16(l1_ref[...], RB2, NC1)


def _iota16():
    return lax.iota(jnp.int32, 16)


def _gather16(v, idx):
    dn = lax.GatherDimensionNumbers(
        offset_dims=(), collapsed_slice_dims=(0,), start_index_map=(0,))
    return lax.gather(v, idx[:, None], dn, (1,),
                      mode=lax.GatherScatterMode.PROMISE_IN_BOUNDS)


def _bf_max(v):
    for sh in (8, 4, 2, 1):
        v = jnp.maximum(v, _gather16(v, _iota16() ^ sh))
    return v


def _bf_min_i32(v):
    for sh in (8, 4, 2, 1):
        v = jnp.minimum(v, _gather16(v, _iota16() ^ sh))
    return v


def _first_set(mask):
    return _bf_min_i32(jnp.where(mask, _iota16(), 16))[0]


def _sc_body(u_hbm, l1_hbm, l2_hbm, z_hbm, uo_hbm, io_hbm,
             datav, l1v, l2v, l3v, uov, iov, zv):
    iota = _iota16()
    wid = lax.axis_index("s") * 2 + lax.axis_index("c")
    row0 = wid * RPW
    pltpu.sync_copy(z_hbm.at[pl.ds(pl.multiple_of(row0, 8), RPW)], zv)

    def row_body(t, _):
        row = row0 + t
        pltpu.sync_copy(u_hbm.at[row], datav)
        pltpu.sync_copy(l1_hbm.at[row], l1v)
        pltpu.sync_copy(l2_hbm.at[row], l2v)

        l3v[pl.ds(0, 16)] = jnp.full((16,), NEG, jnp.float32)
        l3v[pl.ds(16, 16)] = jnp.full((16,), NEG, jnp.float32)

        def build3(i, _):
            nm = _bf_max(l2v[pl.ds(pl.multiple_of(i * 16, 16), 16)])
            base = (i // 16) * 16
            lane = i - base
            v = l3v[pl.ds(pl.multiple_of(base, 16), 16)]
            l3v[pl.ds(pl.multiple_of(base, 16), 16)] = jnp.where(
                iota == lane, nm, v)
            return 0

        lax.fori_loop(0, NC3, build3, 0)

        def ext_body(e, _):
            v3a = l3v[pl.ds(0, 16)]
            v3b = l3v[pl.ds(16, 16)]
            bsel = v3b > v3a
            accv = jnp.where(bsel, v3b, v3a)
            accb = jnp.where(bsel, 16, 0)
            mv = _bf_max(accv)
            cand = jnp.where(accv == mv, iota + accb, 1 << 30)
            j3 = _bf_min_i32(cand)[0]
            v2 = l2v[pl.ds(pl.multiple_of(j3 * 16, 16), 16)]
            l2l = _first_set(v2 == mv)
            j2 = j3 * 16 + l2l
            v1 = l1v[pl.ds(pl.multiple_of(j2 * 16, 16), 16)]
            l1l = _first_set(v1 == mv)
            j1 = j2 * 16 + l1l
            v0 = datav[pl.ds(pl.multiple_of(j1 * 16, 16), 16)]
            l0l = _first_set(v0 == mv)
            g = j1 * 16 + l0l
            ob = (e // 16) * 16
            ol = e - ob
            uo = uov[pl.ds(pl.multiple_of(ob, 16), 16)]
            uov[pl.ds(pl.multiple_of(ob, 16), 16)] = jnp.where(
                iota == ol, mv, uo)
            io = iov[pl.ds(pl.multiple_of(ob, 16), 16)]
            iov[pl.ds(pl.multiple_of(ob, 16), 16)] = jnp.where(
                iota == ol, g, io)
            v0n = jnp.where(iota == l0l, jnp.float32(NEG), v0)
            datav[pl.ds(pl.multiple_of(j1 * 16, 16), 16)] = v0n
            n1 = _bf_max(v0n)
            v1n = jnp.where(iota == l1l, n1, v1)
            l1v[pl.ds(pl.multiple_of(j2 * 16, 16), 16)] = v1n
            n2 = _bf_max(v1n)
            v2n = jnp.where(iota == l2l, n2, v2)
            l2v[pl.ds(pl.multiple_of(j3 * 16, 16), 16)] = v2n
            n3 = _bf_max(v2n)
            b3 = (j3 // 16) * 16
            l3l = j3 - b3
            v3 = l3v[pl.ds(pl.multiple_of(b3, 16), 16)]
            l3v[pl.ds(pl.multiple_of(b3, 16), 16)] = jnp.where(
                iota == l3l, n3, v3)
            return 0

        lax.fori_loop(0, K, ext_body, 0)

        tb = (t // 16) * 16
        vz = zv[pl.ds(pl.multiple_of(tb, 16), 16)]
        zvec = _bf_max(jnp.where(iota == (t - tb), vz, jnp.float32(0.0)))
        for q in range(OUTW // 16):
            uov[pl.ds(q * 16, 16)] = uov[pl.ds(q * 16, 16)] / zvec

        pltpu.sync_copy(uov, uo_hbm.at[row])
        pltpu.sync_copy(iov, io_hbm.at[row])
        return 0

    lax.fori_loop(0, RPW, row_body, 0)


@jax.jit
def kernel(state, W, b):
    f32 = jnp.float32
    wp = jnp.concatenate(
        [W, jnp.zeros((STATE_DIM, NPAD - NUM_ACTIONS), f32)], axis=1)
    bp = jnp.concatenate(
        [b, jnp.full((NPAD - NUM_ACTIONS,), -jnp.inf, f32)]).reshape(1, NPAD)

    m = pl.pallas_call(
        _a0_body,
        grid=(GR, GA),
        in_specs=[
            pl.BlockSpec((RB, STATE_DIM), lambda i, j: (i, 0)),
            pl.BlockSpec((STATE_DIM, AT), lambda i, j: (0, j)),
            pl.BlockSpec((1, AT), lambda i, j: (0, j)),
        ],
        out_specs=pl.BlockSpec((RB, 1), lambda i, j: (i, 0)),
        out_shape=jax.ShapeDtypeStruct((BATCH, 1), f32),
        scratch_shapes=[pltpu.VMEM((RB, 1), f32)],
    )(state, wp, bp)

    u, l1, z = pl.pallas_call(
        _a1_body,
        grid=(GR, GA),
        in_specs=[
            pl.BlockSpec((RB, STATE_DIM), lambda i, j: (i, 0)),
            pl.BlockSpec((STATE_DIM, AT), lambda i, j: (0, j)),
            pl.BlockSpec((1, AT), lambda i, j: (0, j)),
            pl.BlockSpec((RB, 1), lambda i, j: (i, 0)),
        ],
        out_specs=[
            pl.BlockSpec((RB, AT), lambda i, j: (i, j)),
            pl.BlockSpec((RB, AT // 16), lambda i, j: (i, j)),
            pl.BlockSpec((RB, 1), lambda i, j: (i, 0)),
        ],
        out_shape=[
            jax.ShapeDtypeStruct((BATCH, NPAD), f32),
            jax.ShapeDtypeStruct((BATCH, NC1), f32),
            jax.ShapeDtypeStruct((BATCH, 1), f32),
        ],
        scratch_shapes=[pltpu.VMEM((RB, 1), f32)],
    )(state, wp, bp, m)

    l2 = pl.pallas_call(
        _a2_body,
        grid=(BATCH // RB2,),
        in_specs=[pl.BlockSpec((RB2, NC1), lambda i: (i, 0))],
        out_specs=pl.BlockSpec((RB2, NC2), lambda i: (i, 0)),
        out_shape=jax.ShapeDtypeStruct((BATCH, NC2), f32),
    )(l1)

    l2p = jnp.concatenate(
        [l2, jnp.full((BATCH, NC2P - NC2), NEG, f32)], axis=1)

    sc = functools.partial(
        pl.kernel,
        out_type=[
            jax.ShapeDtypeStruct((BATCH, OUTW), f32),
            jax.ShapeDtypeStruct((BATCH, OUTW), jnp.int32),
        ],
        mesh=plsc.VectorSubcoreMesh(core_axis_name="c", subcore_axis_name="s"),
        scratch_types=[
            pltpu.VMEM((NPAD,), f32),
            pltpu.VMEM((NC1,), f32),
            pltpu.VMEM((NC2P,), f32),
            pltpu.VMEM((32,), f32),
            pltpu.VMEM((OUTW,), f32),
            pltpu.VMEM((OUTW,), jnp.int32),
            pltpu.VMEM((RPW,), f32),
        ],
    )(_sc_body)

    uo, io = sc(u, l1, l2p, z.reshape(BATCH))
    return (io[:, :K], uo[:, :K])


# speedup vs baseline: 56.0355x; 2.0797x over previous
import functools

import jax
import jax.numpy as jnp
from jax import lax
from jax.experimental import pallas as pl
from jax.experimental.pallas import tpu as pltpu
from jax.experimental.pallas import tpu_sc as plsc

K = 100
STATE_DIM = 64
NUM_ACTIONS = 100000
BATCH = 1024

AT = 2048
GA = 49
RB = 256
GR = BATCH // RB
NPAD = GA * AT
CHUNK = 128
NC1 = NPAD // CHUNK
NC1P = 896
NC2 = NC1P // 16
OUTW = 112
NWORK = 32
RPW = BATCH // NWORK
NEG = -1.0


def _a0_body(state_ref, w_ref, b_ref, m_ref, ms):
    j = pl.program_id(1)
    s = jnp.dot(state_ref[...], w_ref[...],
                preferred_element_type=jnp.float32) + b_ref[...]
    tm = jnp.max(s, axis=1, keepdims=True)

    @pl.when(j == 0)
    def _():
        ms[...] = tm

    @pl.when(j > 0)
    def _():
        ms[...] = jnp.maximum(ms[...], tm)

    @pl.when(j == GA - 1)
    def _():
        m_ref[...] = ms[...]


def _a1_body(state_ref, w_ref, b_ref, m_ref, u_ref, l1_ref, z_ref,
             zs, l1s, gs):
    j = pl.program_id(1)
    s = jnp.dot(state_ref[...], w_ref[...],
                preferred_element_type=jnp.float32) + b_ref[...]
    u = jnp.exp(s - m_ref[...])
    u_ref[...] = u
    cm = jnp.max(u.reshape(RB, AT // CHUNK, CHUNK), axis=2)
    ts = jnp.sum(u, axis=1, keepdims=True)

    @pl.when(j == 0)
    def _():
        zs[...] = ts

    @pl.when(j > 0)
    def _():
        zs[...] = zs[...] + ts

    tile8 = jnp.concatenate([cm] * 8, axis=1)
    lane = lax.broadcasted_iota(jnp.int32, (RB, 128), 1)
    gpos = j % 8
    prev = jnp.where(jnp.full((RB, 128), gpos == 0),
                     jnp.full((RB, 128), NEG, jnp.float32), gs[...])
    gs[...] = jnp.where((lane // 16) == gpos, tile8, prev)

    @pl.when((gpos == 7) | (j == GA - 1))
    def _():
        l1s[:, pl.ds(pl.multiple_of((j // 8) * 128, 128), 128)] = gs[...]

    @pl.when(j == GA - 1)
    def _():
        z_ref[...] = zs[...]
        l1_ref[...] = l1s[...]


def _iota16():
    return lax.iota(jnp.int32, 16)


def _gather16(v, idx):
    dn = lax.GatherDimensionNumbers(
        offset_dims=(), collapsed_slice_dims=(0,), start_index_map=(0,))
    return lax.gather(v, idx[:, None], dn, (1,),
                      mode=lax.GatherScatterMode.PROMISE_IN_BOUNDS)


def _bf_max(v):
    for sh in (8, 4, 2, 1):
        v = jnp.maximum(v, _gather16(v, _iota16() ^ sh))
    return v


def _bf_min_i32(v):
    for sh in (8, 4, 2, 1):
        v = jnp.minimum(v, _gather16(v, _iota16() ^ sh))
    return v


def _first_set(mask):
    return _bf_min_i32(jnp.where(mask, _iota16(), 16))[0]


def _sc_body(u_hbm, l1_hbm, z_hbm, uo_hbm, io_hbm,
             datav, l1v, l2v, uov, iov, zv):
    iota = _iota16()
    big = jnp.int32(1 << 30)
    wid = lax.axis_index("s") * 2 + lax.axis_index("c")
    row0 = wid * RPW
    pltpu.sync_copy(z_hbm.at[pl.ds(pl.multiple_of(row0, 8), RPW)], zv)

    def row_body(t, _):
        row = row0 + t
        pltpu.sync_copy(u_hbm.at[row], datav)
        pltpu.sync_copy(l1_hbm.at[row], l1v)

        l2v[pl.ds(0, 16)] = jnp.full((16,), NEG, jnp.float32)
        l2v[pl.ds(16, 16)] = jnp.full((16,), NEG, jnp.float32)
        l2v[pl.ds(32, 16)] = jnp.full((16,), NEG, jnp.float32)
        l2v[pl.ds(48, 16)] = jnp.full((16,), NEG, jnp.float32)

        def build2(i, _):
            nm = _bf_max(l1v[pl.ds(pl.multiple_of(i * 16, 16), 16)])
            base = (i // 16) * 16
            lane = i - base
            v = l2v[pl.ds(pl.multiple_of(base, 16), 16)]
            l2v[pl.ds(pl.multiple_of(base, 16), 16)] = jnp.where(
                iota == lane, nm, v)
            return 0

        lax.fori_loop(0, NC2, build2, 0)

        def ext_body(e, _):
            w0 = l2v[pl.ds(0, 16)]
            w1 = l2v[pl.ds(16, 16)]
            w2 = l2v[pl.ds(32, 16)]
            w3 = l2v[pl.ds(48, 16)]
            m01 = jnp.maximum(w0, w1)
            m23 = jnp.maximum(w2, w3)
            accv = jnp.maximum(m01, m23)
            mv = _bf_max(accv)
            c0 = jnp.where(w0 == mv, iota, big)
            c1 = jnp.where(w1 == mv, iota + 16, big)
            c2 = jnp.where(w2 == mv, iota + 32, big)
            c3 = jnp.where(w3 == mv, iota + 48, big)
            cand = jnp.minimum(jnp.minimum(c0, c1), jnp.minimum(c2, c3))
            j2 = _bf_min_i32(cand)[0]
            v1 = l1v[pl.ds(pl.multiple_of(j2 * 16, 16), 16)]
            l1l = _first_set(v1 == mv)
            c = j2 * 16 + l1l
            cbase = c * CHUNK
            acc = jnp.full((16,), 1 << 30, jnp.int32)
            for kk in range(8):
                vk = datav[pl.ds(pl.multiple_of(cbase + kk * 16, 16), 16)]
                acc = jnp.minimum(acc,
                                  jnp.where(vk == mv, iota + kk * 16, big))
            loc = _bf_min_i32(acc)[0]
            g = cbase + loc
            ob = (e // 16) * 16
            ol = e - ob
            uo = uov[pl.ds(pl.multiple_of(ob, 16), 16)]
            uov[pl.ds(pl.multiple_of(ob, 16), 16)] = jnp.where(
                iota == ol, mv, uo)
            io = iov[pl.ds(pl.multiple_of(ob, 16), 16)]
            iov[pl.ds(pl.multiple_of(ob, 16), 16)] = jnp.where(
                iota == ol, g, io)
            lb = (loc // 16) * 16
            ll = loc - lb
            vm = datav[pl.ds(pl.multiple_of(cbase + lb, 16), 16)]
            datav[pl.ds(pl.multiple_of(cbase + lb, 16), 16)] = jnp.where(
                iota == ll, jnp.float32(NEG), vm)
            nmax = jnp.full((16,), NEG, jnp.float32)
            for kk in range(8):
                nmax = jnp.maximum(
                    nmax, datav[pl.ds(pl.multiple_of(cbase + kk * 16, 16),
                                      16)])
            n1 = _bf_max(nmax)
            v1n = jnp.where(iota == l1l, n1, v1)
            l1v[pl.ds(pl.multiple_of(j2 * 16, 16), 16)] = v1n
            n2 = _bf_max(v1n)
            b2 = (j2 // 16) * 16
            l2l = j2 - b2
            v2 = l2v[pl.ds(pl.multiple_of(b2, 16), 16)]
            l2v[pl.ds(pl.multiple_of(b2, 16), 16)] = jnp.where(
                iota == l2l, n2, v2)
            return 0

        lax.fori_loop(0, K, ext_body, 0)

        tb = (t // 16) * 16
        vz = zv[pl.ds(pl.multiple_of(tb, 16), 16)]
        zvec = _bf_max(jnp.where(iota == (t - tb), vz, jnp.float32(0.0)))
        for q in range(OUTW // 16):
            uov[pl.ds(q * 16, 16)] = uov[pl.ds(q * 16, 16)] / zvec

        pltpu.sync_copy(uov, uo_hbm.at[row])
        pltpu.sync_copy(iov, io_hbm.at[row])
        return 0

    lax.fori_loop(0, RPW, row_body, 0)


@jax.jit
def kernel(state, W, b):
    f32 = jnp.float32
    wp = jnp.concatenate(
        [W, jnp.zeros((STATE_DIM, NPAD - NUM_ACTIONS), f32)], axis=1)
    bp = jnp.concatenate(
        [b, jnp.full((NPAD - NUM_ACTIONS,), -jnp.inf, f32)]).reshape(1, NPAD)

    m = pl.pallas_call(
        _a0_body,
        grid=(GR, GA),
        in_specs=[
            pl.BlockSpec((RB, STATE_DIM), lambda i, j: (i, 0)),
            pl.BlockSpec((STATE_DIM, AT), lambda i, j: (0, j)),
            pl.BlockSpec((1, AT), lambda i, j: (0, j)),
        ],
        out_specs=pl.BlockSpec((RB, 1), lambda i, j: (i, 0)),
        out_shape=jax.ShapeDtypeStruct((BATCH, 1), f32),
        scratch_shapes=[pltpu.VMEM((RB, 1), f32)],
    )(state, wp, bp)

    u, l1, z = pl.pallas_call(
        _a1_body,
        grid=(GR, GA),
        in_specs=[
            pl.BlockSpec((RB, STATE_DIM), lambda i, j: (i, 0)),
            pl.BlockSpec((STATE_DIM, AT), lambda i, j: (0, j)),
            pl.BlockSpec((1, AT), lambda i, j: (0, j)),
            pl.BlockSpec((RB, 1), lambda i, j: (i, 0)),
        ],
        out_specs=[
            pl.BlockSpec((RB, AT), lambda i, j: (i, j)),
            pl.BlockSpec((RB, NC1P), lambda i, j: (i, 0)),
            pl.BlockSpec((RB, 1), lambda i, j: (i, 0)),
        ],
        out_shape=[
            jax.ShapeDtypeStruct((BATCH, NPAD), f32),
            jax.ShapeDtypeStruct((BATCH, NC1P), f32),
            jax.ShapeDtypeStruct((BATCH, 1), f32),
        ],
        scratch_shapes=[pltpu.VMEM((RB, 1), f32),
                        pltpu.VMEM((RB, NC1P), f32),
                        pltpu.VMEM((RB, 128), f32)],
    )(state, wp, bp, m)

    sc = functools.partial(
        pl.kernel,
        out_type=[
            jax.ShapeDtypeStruct((BATCH, OUTW), f32),
            jax.ShapeDtypeStruct((BATCH, OUTW), jnp.int32),
        ],
        mesh=plsc.VectorSubcoreMesh(core_axis_name="c", subcore_axis_name="s"),
        scratch_types=[
            pltpu.VMEM((NPAD,), f32),
            pltpu.VMEM((NC1P,), f32),
            pltpu.VMEM((64,), f32),
            pltpu.VMEM((OUTW,), f32),
            pltpu.VMEM((OUTW,), jnp.int32),
            pltpu.VMEM((RPW,), f32),
        ],
    )(_sc_body)

    uo, io = sc(u, l1, z.reshape(BATCH))
    return (io[:, :K], uo[:, :K])


# speedup vs baseline: 73.6732x; 1.3148x over previous
import functools

import jax
import jax.numpy as jnp
from jax import lax
from jax.experimental import pallas as pl
from jax.experimental.pallas import tpu as pltpu
from jax.experimental.pallas import tpu_sc as plsc

K = 100
STATE_DIM = 64
NUM_ACTIONS = 100000
BATCH = 1024

AT = 2048
GA = 49
RB = 256
GR = BATCH // RB
NPAD = GA * AT
CHUNK = 128
NC1 = NPAD // CHUNK
NC1P = 896
NC2 = NC1P // 16
OUTW = 112
NWORK = 32
RPW = RB // NWORK
NEG = -1.0


def _a0_body(state_ref, w_ref, b_ref, m_ref, ms):
    j = pl.program_id(1)
    s = jnp.dot(state_ref[...], w_ref[...],
                preferred_element_type=jnp.float32) + b_ref[...]
    tm = jnp.max(s, axis=1, keepdims=True)

    @pl.when(j == 0)
    def _():
        ms[...] = tm

    @pl.when(j > 0)
    def _():
        ms[...] = jnp.maximum(ms[...], tm)

    @pl.when(j == GA - 1)
    def _():
        m_ref[...] = ms[...]


def _a1_body(state_ref, w_ref, b_ref, m_ref, u_ref, l1_ref, z_ref,
             zs, l1s, gs):
    j = pl.program_id(1)
    s = jnp.dot(state_ref[...], w_ref[...],
                preferred_element_type=jnp.float32) + b_ref[...]
    u = jnp.exp(s - m_ref[...])
    u_ref[...] = u
    cm = jnp.max(u.reshape(RB, AT // CHUNK, CHUNK), axis=2)
    ts = jnp.sum(u, axis=1, keepdims=True)

    @pl.when(j == 0)
    def _():
        zs[...] = ts

    @pl.when(j > 0)
    def _():
        zs[...] = zs[...] + ts

    tile8 = jnp.concatenate([cm] * 8, axis=1)
    lane = lax.broadcasted_iota(jnp.int32, (RB, 128), 1)
    gpos = j % 8
    prev = jnp.where(jnp.full((RB, 128), gpos == 0),
                     jnp.full((RB, 128), NEG, jnp.float32), gs[...])
    gs[...] = jnp.where((lane // 16) == gpos, tile8, prev)

    @pl.when((gpos == 7) | (j == GA - 1))
    def _():
        l1s[:, pl.ds(pl.multiple_of((j // 8) * 128, 128), 128)] = gs[...]

    @pl.when(j == GA - 1)
    def _():
        z_ref[...] = zs[...]
        l1_ref[...] = l1s[...]


def _iota16():
    return lax.iota(jnp.int32, 16)


def _gather16(v, idx):
    dn = lax.GatherDimensionNumbers(
        offset_dims=(), collapsed_slice_dims=(0,), start_index_map=(0,))
    return lax.gather(v, idx[:, None], dn, (1,),
                      mode=lax.GatherScatterMode.PROMISE_IN_BOUNDS)


def _bf_max(v):
    for sh in (8, 4, 2, 1):
        v = jnp.maximum(v, _gather16(v, _iota16() ^ sh))
    return v


def _bf_min_i32(v):
    for sh in (8, 4, 2, 1):
        v = jnp.minimum(v, _gather16(v, _iota16() ^ sh))
    return v


def _first_set(mask):
    return _bf_min_i32(jnp.where(mask, _iota16(), 16))[0]


def _sc_body(u_hbm, l1_hbm, z_hbm, uo_hbm, io_hbm,
             datav, l1v, l2v, uov, iov, zv):
    iota = _iota16()
    big = jnp.int32(1 << 30)
    wid = lax.axis_index("s") * 2 + lax.axis_index("c")
    row0 = wid * RPW
    pltpu.sync_copy(z_hbm.at[pl.ds(pl.multiple_of(row0, 8), RPW)],
                    zv.at[pl.ds(0, RPW)])

    def row_body(t, _):
        row = row0 + t
        pltpu.sync_copy(u_hbm.at[row], datav)
        pltpu.sync_copy(l1_hbm.at[row], l1v)

        l2v[pl.ds(0, 16)] = jnp.full((16,), NEG, jnp.float32)
        l2v[pl.ds(16, 16)] = jnp.full((16,), NEG, jnp.float32)
        l2v[pl.ds(32, 16)] = jnp.full((16,), NEG, jnp.float32)
        l2v[pl.ds(48, 16)] = jnp.full((16,), NEG, jnp.float32)

        def build2(i, _):
            nm = _bf_max(l1v[pl.ds(pl.multiple_of(i * 16, 16), 16)])
            base = (i // 16) * 16
            lane = i - base
            v = l2v[pl.ds(pl.multiple_of(base, 16), 16)]
            l2v[pl.ds(pl.multiple_of(base, 16), 16)] = jnp.where(
                iota == lane, nm, v)
            return 0

        lax.fori_loop(0, NC2, build2, 0)

        def ext_body(e, _):
            w0 = l2v[pl.ds(0, 16)]
            w1 = l2v[pl.ds(16, 16)]
            w2 = l2v[pl.ds(32, 16)]
            w3 = l2v[pl.ds(48, 16)]
            m01 = jnp.maximum(w0, w1)
            m23 = jnp.maximum(w2, w3)
            accv = jnp.maximum(m01, m23)
            mv = _bf_max(accv)
            c0 = jnp.where(w0 == mv, iota, big)
            c1 = jnp.where(w1 == mv, iota + 16, big)
            c2 = jnp.where(w2 == mv, iota + 32, big)
            c3 = jnp.where(w3 == mv, iota + 48, big)
            cand = jnp.minimum(jnp.minimum(c0, c1), jnp.minimum(c2, c3))
            j2 = _bf_min_i32(cand)[0]
            v1 = l1v[pl.ds(pl.multiple_of(j2 * 16, 16), 16)]
            l1l = _first_set(v1 == mv)
            c = j2 * 16 + l1l
            cbase = c * CHUNK
            acc = jnp.full((16,), 1 << 30, jnp.int32)
            for kk in range(8):
                vk = datav[pl.ds(pl.multiple_of(cbase + kk * 16, 16), 16)]
                acc = jnp.minimum(acc,
                                  jnp.where(vk == mv, iota + kk * 16, big))
            loc = _bf_min_i32(acc)[0]
            g = cbase + loc
            ob = (e // 16) * 16
            ol = e - ob
            uo = uov[pl.ds(pl.multiple_of(ob, 16), 16)]
            uov[pl.ds(pl.multiple_of(ob, 16), 16)] = jnp.where(
                iota == ol, mv, uo)
            io = iov[pl.ds(pl.multiple_of(ob, 16), 16)]
            iov[pl.ds(pl.multiple_of(ob, 16), 16)] = jnp.where(
                iota == ol, g, io)
            lb = (loc // 16) * 16
            ll = loc - lb
            vm = datav[pl.ds(pl.multiple_of(cbase + lb, 16), 16)]
            datav[pl.ds(pl.multiple_of(cbase + lb, 16), 16)] = jnp.where(
                iota == ll, jnp.float32(NEG), vm)
            nmax = jnp.full((16,), NEG, jnp.float32)
            for kk in range(8):
                nmax = jnp.maximum(
                    nmax, datav[pl.ds(pl.multiple_of(cbase + kk * 16, 16),
                                      16)])
            n1 = _bf_max(nmax)
            v1n = jnp.where(iota == l1l, n1, v1)
            l1v[pl.ds(pl.multiple_of(j2 * 16, 16), 16)] = v1n
            n2 = _bf_max(v1n)
            b2 = (j2 // 16) * 16
            l2l = j2 - b2
            v2 = l2v[pl.ds(pl.multiple_of(b2, 16), 16)]
            l2v[pl.ds(pl.multiple_of(b2, 16), 16)] = jnp.where(
                iota == l2l, n2, v2)
            return 0

        lax.fori_loop(0, K, ext_body, 0)

        tb = (t // 16) * 16
        vz = zv[pl.ds(pl.multiple_of(tb, 16), 16)]
        zvec = _bf_max(jnp.where(iota == (t - tb), vz, jnp.float32(0.0)))
        for q in range(OUTW // 16):
            uov[pl.ds(q * 16, 16)] = uov[pl.ds(q * 16, 16)] / zvec

        pltpu.sync_copy(uov, uo_hbm.at[row])
        pltpu.sync_copy(iov, io_hbm.at[row])
        return 0

    lax.fori_loop(0, RPW, row_body, 0)


@jax.jit
def kernel(state, W, b):
    f32 = jnp.float32
    wp = jnp.concatenate(
        [W, jnp.zeros((STATE_DIM, NPAD - NUM_ACTIONS), f32)], axis=1)
    bp = jnp.concatenate(
        [b, jnp.full((NPAD - NUM_ACTIONS,), -jnp.inf, f32)]).reshape(1, NPAD)

    m = pl.pallas_call(
        _a0_body,
        grid=(GR, GA),
        in_specs=[
            pl.BlockSpec((RB, STATE_DIM), lambda i, j: (i, 0)),
            pl.BlockSpec((STATE_DIM, AT), lambda i, j: (0, j)),
            pl.BlockSpec((1, AT), lambda i, j: (0, j)),
        ],
        out_specs=pl.BlockSpec((RB, 1), lambda i, j: (i, 0)),
        out_shape=jax.ShapeDtypeStruct((BATCH, 1), f32),
        scratch_shapes=[pltpu.VMEM((RB, 1), f32)],
    )(state, wp, bp)

    a1 = functools.partial(
        pl.pallas_call,
        _a1_body,
        grid=(1, GA),
        in_specs=[
            pl.BlockSpec((RB, STATE_DIM), lambda i, j: (0, 0)),
            pl.BlockSpec((STATE_DIM, AT), lambda i, j: (0, j)),
            pl.BlockSpec((1, AT), lambda i, j: (0, j)),
            pl.BlockSpec((RB, 1), lambda i, j: (0, 0)),
        ],
        out_specs=[
            pl.BlockSpec((RB, AT), lambda i, j: (0, j)),
            pl.BlockSpec((RB, NC1P), lambda i, j: (0, 0)),
            pl.BlockSpec((RB, 1), lambda i, j: (0, 0)),
        ],
        out_shape=[
            jax.ShapeDtypeStruct((RB, NPAD), f32),
            jax.ShapeDtypeStruct((RB, NC1P), f32),
            jax.ShapeDtypeStruct((RB, 1), f32),
        ],
        scratch_shapes=[pltpu.VMEM((RB, 1), f32),
                        pltpu.VMEM((RB, NC1P), f32),
                        pltpu.VMEM((RB, 128), f32)],
    )

    sc = functools.partial(
        pl.kernel,
        out_type=[
            jax.ShapeDtypeStruct((RB, OUTW), f32),
            jax.ShapeDtypeStruct((RB, OUTW), jnp.int32),
        ],
        mesh=plsc.VectorSubcoreMesh(core_axis_name="c", subcore_axis_name="s"),
        scratch_types=[
            pltpu.VMEM((NPAD,), f32),
            pltpu.VMEM((NC1P,), f32),
            pltpu.VMEM((64,), f32),
            pltpu.VMEM((OUTW,), f32),
            pltpu.VMEM((OUTW,), jnp.int32),
            pltpu.VMEM((16,), f32),
        ],
    )(_sc_body)

    uos, ios = [], []
    for blk in range(GR):
        state_b = lax.slice_in_dim(state, blk * RB, (blk + 1) * RB, axis=0)
        m_b = lax.slice_in_dim(m, blk * RB, (blk + 1) * RB, axis=0)
        u, l1, z = a1()(state_b, wp, bp, m_b)
        uo, io = sc(u, l1, z.reshape(RB))
        uos.append(uo)
        ios.append(io)
    uo = jnp.concatenate(uos, axis=0)
    io = jnp.concatenate(ios, axis=0)
    return (io[:, :K], uo[:, :K])


# speedup vs baseline: 73.9099x; 1.0032x over previous
import functools

import jax
import jax.numpy as jnp
from jax import lax
from jax.experimental import pallas as pl
from jax.experimental.pallas import tpu as pltpu
from jax.experimental.pallas import tpu_sc as plsc

K = 100
STATE_DIM = 64
NUM_ACTIONS = 100000
BATCH = 1024

AT = 2048
GA = 49
RB0 = 256
RB = 128
GR = BATCH // RB
NPAD = GA * AT
CHUNK = 128
NC1 = NPAD // CHUNK
NC1P = 896
NC2 = NC1P // 16
OUTW = 112
NWORK = 32
RPW = RB // NWORK
NEG = -1.0


def _a0_body(state_ref, w_ref, b_ref, m_ref, ms):
    j = pl.program_id(1)
    s = jnp.dot(state_ref[...], w_ref[...],
                preferred_element_type=jnp.float32) + b_ref[...]
    tm = jnp.max(s, axis=1, keepdims=True)

    @pl.when(j == 0)
    def _():
        ms[...] = tm

    @pl.when(j > 0)
    def _():
        ms[...] = jnp.maximum(ms[...], tm)

    @pl.when(j == GA - 1)
    def _():
        m_ref[...] = ms[...]


def _a1_body(state_ref, w_ref, b_ref, m_ref, u_ref, l1_ref, z_ref,
             zs, l1s, gs):
    j = pl.program_id(1)
    s = jnp.dot(state_ref[...], w_ref[...],
                preferred_element_type=jnp.float32) + b_ref[...]
    u = jnp.exp(s - m_ref[...])
    u_ref[...] = u
    cm = jnp.max(u.reshape(RB, AT // CHUNK, CHUNK), axis=2)
    ts = jnp.sum(u, axis=1, keepdims=True)

    @pl.when(j == 0)
    def _():
        zs[...] = ts

    @pl.when(j > 0)
    def _():
        zs[...] = zs[...] + ts

    tile8 = jnp.concatenate([cm] * 8, axis=1)
    lane = lax.broadcasted_iota(jnp.int32, (RB, 128), 1)
    gpos = j % 8
    prev = jnp.where(jnp.full((RB, 128), gpos == 0),
                     jnp.full((RB, 128), NEG, jnp.float32), gs[...])
    gs[...] = jnp.where((lane // 16) == gpos, tile8, prev)

    @pl.when((gpos == 7) | (j == GA - 1))
    def _():
        l1s[:, pl.ds(pl.multiple_of((j // 8) * 128, 128), 128)] = gs[...]

    @pl.when(j == GA - 1)
    def _():
        z_ref[...] = zs[...]
        l1_ref[...] = l1s[...]


def _iota16():
    return lax.iota(jnp.int32, 16)


def _gather16(v, idx):
    dn = lax.GatherDimensionNumbers(
        offset_dims=(), collapsed_slice_dims=(0,), start_index_map=(0,))
    return lax.gather(v, idx[:, None], dn, (1,),
                      mode=lax.GatherScatterMode.PROMISE_IN_BOUNDS)


def _bf_max(v):
    for sh in (8, 4, 2, 1):
        v = jnp.maximum(v, _gather16(v, _iota16() ^ sh))
    return v


def _bf_min_i32(v):
    for sh in (8, 4, 2, 1):
        v = jnp.minimum(v, _gather16(v, _iota16() ^ sh))
    return v


def _first_set(mask):
    return _bf_min_i32(jnp.where(mask, _iota16(), 16))[0]


def _sc_body(u_hbm, l1_hbm, z_hbm, uo_hbm, io_hbm,
             datav, l1v, l2v, uov, iov, zv):
    iota = _iota16()
    big = jnp.int32(1 << 30)
    wid = lax.axis_index("s") * 2 + lax.axis_index("c")
    row0 = wid * RPW
    pltpu.sync_copy(z_hbm, zv)

    def row_body(t, _):
        row = row0 + t
        pltpu.sync_copy(u_hbm.at[row], datav)
        pltpu.sync_copy(l1_hbm.at[row], l1v)

        l2v[pl.ds(0, 16)] = jnp.full((16,), NEG, jnp.float32)
        l2v[pl.ds(16, 16)] = jnp.full((16,), NEG, jnp.float32)
        l2v[pl.ds(32, 16)] = jnp.full((16,), NEG, jnp.float32)
        l2v[pl.ds(48, 16)] = jnp.full((16,), NEG, jnp.float32)

        def build2(i, _):
            nm = _bf_max(l1v[pl.ds(pl.multiple_of(i * 16, 16), 16)])
            base = (i // 16) * 16
            lane = i - base
            v = l2v[pl.ds(pl.multiple_of(base, 16), 16)]
            l2v[pl.ds(pl.multiple_of(base, 16), 16)] = jnp.where(
                iota == lane, nm, v)
            return 0

        lax.fori_loop(0, NC2, build2, 0)

        def ext_body(e, _):
            w0 = l2v[pl.ds(0, 16)]
            w1 = l2v[pl.ds(16, 16)]
            w2 = l2v[pl.ds(32, 16)]
            w3 = l2v[pl.ds(48, 16)]
            m01 = jnp.maximum(w0, w1)
            m23 = jnp.maximum(w2, w3)
            accv = jnp.maximum(m01, m23)
            mv = _bf_max(accv)
            c0 = jnp.where(w0 == mv, iota, big)
            c1 = jnp.where(w1 == mv, iota + 16, big)
            c2 = jnp.where(w2 == mv, iota + 32, big)
            c3 = jnp.where(w3 == mv, iota + 48, big)
            cand = jnp.minimum(jnp.minimum(c0, c1), jnp.minimum(c2, c3))
            j2 = _bf_min_i32(cand)[0]
            v1 = l1v[pl.ds(pl.multiple_of(j2 * 16, 16), 16)]
            l1l = _first_set(v1 == mv)
            c = j2 * 16 + l1l
            cbase = c * CHUNK
            acc = jnp.full((16,), 1 << 30, jnp.int32)
            for kk in range(8):
                vk = datav[pl.ds(pl.multiple_of(cbase + kk * 16, 16), 16)]
                acc = jnp.minimum(acc,
                                  jnp.where(vk == mv, iota + kk * 16, big))
            loc = _bf_min_i32(acc)[0]
            g = cbase + loc
            ob = (e // 16) * 16
            ol = e - ob
            uo = uov[pl.ds(pl.multiple_of(ob, 16), 16)]
            uov[pl.ds(pl.multiple_of(ob, 16), 16)] = jnp.where(
                iota == ol, mv, uo)
            io = iov[pl.ds(pl.multiple_of(ob, 16), 16)]
            iov[pl.ds(pl.multiple_of(ob, 16), 16)] = jnp.where(
                iota == ol, g, io)
            lb = (loc // 16) * 16
            ll = loc - lb
            vm = datav[pl.ds(pl.multiple_of(cbase + lb, 16), 16)]
            datav[pl.ds(pl.multiple_of(cbase + lb, 16), 16)] = jnp.where(
                iota == ll, jnp.float32(NEG), vm)
            nmax = jnp.full((16,), NEG, jnp.float32)
            for kk in range(8):
                nmax = jnp.maximum(
                    nmax, datav[pl.ds(pl.multiple_of(cbase + kk * 16, 16),
                                      16)])
            n1 = _bf_max(nmax)
            v1n = jnp.where(iota == l1l, n1, v1)
            l1v[pl.ds(pl.multiple_of(j2 * 16, 16), 16)] = v1n
            n2 = _bf_max(v1n)
            b2 = (j2 // 16) * 16
            l2l = j2 - b2
            v2 = l2v[pl.ds(pl.multiple_of(b2, 16), 16)]
            l2v[pl.ds(pl.multiple_of(b2, 16), 16)] = jnp.where(
                iota == l2l, n2, v2)
            return 0

        lax.fori_loop(0, K, ext_body, 0)

        tb = (row // 16) * 16
        vz = zv[pl.ds(pl.multiple_of(tb, 16), 16)]
        zvec = _bf_max(jnp.where(iota == (row - tb), vz, jnp.float32(0.0)))
        for q in range(OUTW // 16):
            uov[pl.ds(q * 16, 16)] = uov[pl.ds(q * 16, 16)] / zvec

        pltpu.sync_copy(uov, uo_hbm.at[row])
        pltpu.sync_copy(iov, io_hbm.at[row])
        return 0

    lax.fori_loop(0, RPW, row_body, 0)


@jax.jit
def kernel(state, W, b):
    f32 = jnp.float32
    wp = jnp.concatenate(
        [W, jnp.zeros((STATE_DIM, NPAD - NUM_ACTIONS), f32)], axis=1)
    bp = jnp.concatenate(
        [b, jnp.full((NPAD - NUM_ACTIONS,), -jnp.inf, f32)]).reshape(1, NPAD)

    m = pl.pallas_call(
        _a0_body,
        grid=(BATCH // RB0, GA),
        in_specs=[
            pl.BlockSpec((RB0, STATE_DIM), lambda i, j: (i, 0)),
            pl.BlockSpec((STATE_DIM, AT), lambda i, j: (0, j)),
            pl.BlockSpec((1, AT), lambda i, j: (0, j)),
        ],
        out_specs=pl.BlockSpec((RB0, 1), lambda i, j: (i, 0)),
        out_shape=jax.ShapeDtypeStruct((BATCH, 1), f32),
        scratch_shapes=[pltpu.VMEM((RB0, 1), f32)],
    )(state, wp, bp)

    a1 = functools.partial(
        pl.pallas_call,
        _a1_body,
        grid=(1, GA),
        in_specs=[
            pl.BlockSpec((RB, STATE_DIM), lambda i, j: (0, 0)),
            pl.BlockSpec((STATE_DIM, AT), lambda i, j: (0, j)),
            pl.BlockSpec((1, AT), lambda i, j: (0, j)),
            pl.BlockSpec((RB, 1), lambda i, j: (0, 0)),
        ],
        out_specs=[
            pl.BlockSpec((RB, AT), lambda i, j: (0, j)),
            pl.BlockSpec((RB, NC1P), lambda i, j: (0, 0)),
            pl.BlockSpec((RB, 1), lambda i, j: (0, 0)),
        ],
        out_shape=[
            jax.ShapeDtypeStruct((RB, NPAD), f32),
            jax.ShapeDtypeStruct((RB, NC1P), f32),
            jax.ShapeDtypeStruct((RB, 1), f32),
        ],
        scratch_shapes=[pltpu.VMEM((RB, 1), f32),
                        pltpu.VMEM((RB, NC1P), f32),
                        pltpu.VMEM((RB, 128), f32)],
    )

    sc = functools.partial(
        pl.kernel,
        out_type=[
            jax.ShapeDtypeStruct((RB, OUTW), f32),
            jax.ShapeDtypeStruct((RB, OUTW), jnp.int32),
        ],
        mesh=plsc.VectorSubcoreMesh(core_axis_name="c", subcore_axis_name="s"),
        scratch_types=[
            pltpu.VMEM((NPAD,), f32),
            pltpu.VMEM((NC1P,), f32),
            pltpu.VMEM((64,), f32),
            pltpu.VMEM((OUTW,), f32),
            pltpu.VMEM((OUTW,), jnp.int32),
            pltpu.VMEM((RB,), f32),
        ],
    )(_sc_body)

    uos, ios = [], []
    for blk in range(GR):
        state_b = lax.slice_in_dim(state, blk * RB, (blk + 1) * RB, axis=0)
        m_b = lax.slice_in_dim(m, blk * RB, (blk + 1) * RB, axis=0)
        u, l1, z = a1()(state_b, wp, bp, m_b)
        uo, io = sc(u, l1, z.reshape(RB))
        uos.append(uo)
        ios.append(io)
    uo = jnp.concatenate(uos, axis=0)
    io = jnp.concatenate(ios, axis=0)
    return (io[:, :K], uo[:, :K])


# speedup vs baseline: 78.7497x; 1.0655x over previous
import functools

import jax
import jax.numpy as jnp
from jax import lax
from jax.experimental import pallas as pl
from jax.experimental.pallas import tpu as pltpu
from jax.experimental.pallas import tpu_sc as plsc

K = 100
STATE_DIM = 64
NUM_ACTIONS = 100000
BATCH = 1024

AT = 2048
GA = 49
RB0 = 256
RB = 128
GR = BATCH // RB
NPAD = GA * AT
CHUNK = 128
NC1 = NPAD // CHUNK
NC1P = 896
NC2 = NC1P // 16
OUTW = 112
NWORK = 32
RPW = RB // NWORK
NEG = -1.0


def _a0_body(state_ref, w_ref, b_ref, m_ref, ms):
    j = pl.program_id(1)
    s = jnp.dot(state_ref[...], w_ref[...],
                preferred_element_type=jnp.float32) + b_ref[...]
    tm = jnp.max(s, axis=1, keepdims=True)

    @pl.when(j == 0)
    def _():
        ms[...] = tm

    @pl.when(j > 0)
    def _():
        ms[...] = jnp.maximum(ms[...], tm)

    @pl.when(j == GA - 1)
    def _():
        m_ref[...] = ms[...]


def _a1_body(state_ref, w_ref, b_ref, m_ref, u_ref, l1_ref, z_ref,
             zs, l1s, gs):
    j = pl.program_id(1)
    s = jnp.dot(state_ref[...], w_ref[...],
                preferred_element_type=jnp.float32) + b_ref[...]
    u = jnp.exp(s - m_ref[...])
    u_ref[...] = u
    cm = jnp.max(u.reshape(RB, AT // CHUNK, CHUNK), axis=2)
    ts = jnp.sum(u, axis=1, keepdims=True)

    @pl.when(j == 0)
    def _():
        zs[...] = ts

    @pl.when(j > 0)
    def _():
        zs[...] = zs[...] + ts

    tile8 = jnp.concatenate([cm] * 8, axis=1)
    lane = lax.broadcasted_iota(jnp.int32, (RB, 128), 1)
    gpos = j % 8
    prev = jnp.where(jnp.full((RB, 128), gpos == 0),
                     jnp.full((RB, 128), NEG, jnp.float32), gs[...])
    gs[...] = jnp.where((lane // 16) == gpos, tile8, prev)

    @pl.when((gpos == 7) | (j == GA - 1))
    def _():
        l1s[:, pl.ds(pl.multiple_of((j // 8) * 128, 128), 128)] = gs[...]

    @pl.when(j == GA - 1)
    def _():
        z_ref[...] = zs[...]
        l1_ref[...] = l1s[...]


def _iota16():
    return lax.iota(jnp.int32, 16)


def _gather16(v, idx):
    dn = lax.GatherDimensionNumbers(
        offset_dims=(), collapsed_slice_dims=(0,), start_index_map=(0,))
    return lax.gather(v, idx[:, None], dn, (1,),
                      mode=lax.GatherScatterMode.PROMISE_IN_BOUNDS)


def _bf_max(v):
    for sh in (8, 4, 2, 1):
        v = jnp.maximum(v, _gather16(v, _iota16() ^ sh))
    return v


def _bf_min_i32(v):
    for sh in (8, 4, 2, 1):
        v = jnp.minimum(v, _gather16(v, _iota16() ^ sh))
    return v


def _first_set(mask):
    return _bf_min_i32(jnp.where(mask, _iota16(), 16))[0]


def _sc_body(u_hbm, l1_hbm, z_hbm, uo_hbm, io_hbm,
             datav, l1v, l2v, uov, iov, zv):
    iota = _iota16()
    big = jnp.int32(1 << 30)
    wid = lax.axis_index("s") * 2 + lax.axis_index("c")
    row0 = wid * RPW
    pltpu.sync_copy(z_hbm, zv)

    def row_body(t, _):
        row = row0 + t
        pltpu.sync_copy(u_hbm.at[row], datav)
        pltpu.sync_copy(l1_hbm.at[row], l1v)

        l2v[pl.ds(0, 16)] = jnp.full((16,), NEG, jnp.float32)
        l2v[pl.ds(16, 16)] = jnp.full((16,), NEG, jnp.float32)
        l2v[pl.ds(32, 16)] = jnp.full((16,), NEG, jnp.float32)
        l2v[pl.ds(48, 16)] = jnp.full((16,), NEG, jnp.float32)

        def build2(i, _):
            nm = _bf_max(l1v[pl.ds(pl.multiple_of(i * 16, 16), 16)])
            base = (i // 16) * 16
            lane = i - base
            v = l2v[pl.ds(pl.multiple_of(base, 16), 16)]
            l2v[pl.ds(pl.multiple_of(base, 16), 16)] = jnp.where(
                iota == lane, nm, v)
            return 0

        lax.fori_loop(0, NC2, build2, 0)

        def ext_body(e, _):
            w0 = l2v[pl.ds(0, 16)]
            w1 = l2v[pl.ds(16, 16)]
            w2 = l2v[pl.ds(32, 16)]
            w3 = l2v[pl.ds(48, 16)]
            m01 = jnp.maximum(w0, w1)
            m23 = jnp.maximum(w2, w3)
            accv = jnp.maximum(m01, m23)
            mv = _bf_max(accv)
            c0 = jnp.where(w0 == mv, iota, big)
            c1 = jnp.where(w1 == mv, iota + 16, big)
            c2 = jnp.where(w2 == mv, iota + 32, big)
            c3 = jnp.where(w3 == mv, iota + 48, big)
            cand = jnp.minimum(jnp.minimum(c0, c1), jnp.minimum(c2, c3))
            j2 = _bf_min_i32(cand)[0]
            v1 = l1v[pl.ds(pl.multiple_of(j2 * 16, 16), 16)]
            l1l = _first_set(v1 == mv)
            c = j2 * 16 + l1l
            cbase = c * CHUNK
            acc = jnp.full((16,), 1 << 30, jnp.int32)
            vks = []
            for kk in range(8):
                vk = datav[pl.ds(pl.multiple_of(cbase + kk * 16, 16), 16)]
                vks.append(vk)
                acc = jnp.minimum(acc,
                                  jnp.where(vk == mv, iota + kk * 16, big))
            locv = _bf_min_i32(acc)
            loc = locv[0]
            g = cbase + loc
            ob = (e // 16) * 16
            ol = e - ob
            uo = uov[pl.ds(pl.multiple_of(ob, 16), 16)]
            uov[pl.ds(pl.multiple_of(ob, 16), 16)] = jnp.where(
                iota == ol, mv, uo)
            io = iov[pl.ds(pl.multiple_of(ob, 16), 16)]
            iov[pl.ds(pl.multiple_of(ob, 16), 16)] = jnp.where(
                iota == ol, g, io)
            nmax = jnp.full((16,), NEG, jnp.float32)
            for kk in range(8):
                vkm = jnp.where(iota + kk * 16 == locv, jnp.float32(NEG),
                                vks[kk])
                datav[pl.ds(pl.multiple_of(cbase + kk * 16, 16), 16)] = vkm
                nmax = jnp.maximum(nmax, vkm)
            n1 = _bf_max(nmax)
            v1n = jnp.where(iota == l1l, n1, v1)
            l1v[pl.ds(pl.multiple_of(j2 * 16, 16), 16)] = v1n
            n2 = _bf_max(v1n)
            b2 = (j2 // 16) * 16
            l2l = j2 - b2
            v2 = l2v[pl.ds(pl.multiple_of(b2, 16), 16)]
            l2v[pl.ds(pl.multiple_of(b2, 16), 16)] = jnp.where(
                iota == l2l, n2, v2)
            return 0

        lax.fori_loop(0, K, ext_body, 0)

        tb = (row // 16) * 16
        vz = zv[pl.ds(pl.multiple_of(tb, 16), 16)]
        zvec = _bf_max(jnp.where(iota == (row - tb), vz, jnp.float32(0.0)))
        for q in range(OUTW // 16):
            uov[pl.ds(q * 16, 16)] = uov[pl.ds(q * 16, 16)] / zvec

        pltpu.sync_copy(uov, uo_hbm.at[row])
        pltpu.sync_copy(iov, io_hbm.at[row])
        return 0

    lax.fori_loop(0, RPW, row_body, 0)


@jax.jit
def kernel(state, W, b):
    f32 = jnp.float32
    wp = jnp.concatenate(
        [W, jnp.zeros((STATE_DIM, NPAD - NUM_ACTIONS), f32)], axis=1)
    bp = jnp.concatenate(
        [b, jnp.full((NPAD - NUM_ACTIONS,), -jnp.inf, f32)]).reshape(1, NPAD)

    m = pl.pallas_call(
        _a0_body,
        grid=(BATCH // RB0, GA),
        in_specs=[
            pl.BlockSpec((RB0, STATE_DIM), lambda i, j: (i, 0)),
            pl.BlockSpec((STATE_DIM, AT), lambda i, j: (0, j)),
            pl.BlockSpec((1, AT), lambda i, j: (0, j)),
        ],
        out_specs=pl.BlockSpec((RB0, 1), lambda i, j: (i, 0)),
        out_shape=jax.ShapeDtypeStruct((BATCH, 1), f32),
        scratch_shapes=[pltpu.VMEM((RB0, 1), f32)],
    )(state, wp, bp)

    a1 = functools.partial(
        pl.pallas_call,
        _a1_body,
        grid=(1, GA),
        in_specs=[
            pl.BlockSpec((RB, STATE_DIM), lambda i, j: (0, 0)),
            pl.BlockSpec((STATE_DIM, AT), lambda i, j: (0, j)),
            pl.BlockSpec((1, AT), lambda i, j: (0, j)),
            pl.BlockSpec((RB, 1), lambda i, j: (0, 0)),
        ],
        out_specs=[
            pl.BlockSpec((RB, AT), lambda i, j: (0, j)),
            pl.BlockSpec((RB, NC1P), lambda i, j: (0, 0)),
            pl.BlockSpec((RB, 1), lambda i, j: (0, 0)),
        ],
        out_shape=[
            jax.ShapeDtypeStruct((RB, NPAD), f32),
            jax.ShapeDtypeStruct((RB, NC1P), f32),
            jax.ShapeDtypeStruct((RB, 1), f32),
        ],
        scratch_shapes=[pltpu.VMEM((RB, 1), f32),
                        pltpu.VMEM((RB, NC1P), f32),
                        pltpu.VMEM((RB, 128), f32)],
    )

    sc = functools.partial(
        pl.kernel,
        out_type=[
            jax.ShapeDtypeStruct((RB, OUTW), f32),
            jax.ShapeDtypeStruct((RB, OUTW), jnp.int32),
        ],
        mesh=plsc.VectorSubcoreMesh(core_axis_name="c", subcore_axis_name="s"),
        scratch_types=[
            pltpu.VMEM((NPAD,), f32),
            pltpu.VMEM((NC1P,), f32),
            pltpu.VMEM((64,), f32),
            pltpu.VMEM((OUTW,), f32),
            pltpu.VMEM((OUTW,), jnp.int32),
            pltpu.VMEM((RB,), f32),
        ],
    )(_sc_body)

    uos, ios = [], []
    for blk in range(GR):
        state_b = lax.slice_in_dim(state, blk * RB, (blk + 1) * RB, axis=0)
        m_b = lax.slice_in_dim(m, blk * RB, (blk + 1) * RB, axis=0)
        u, l1, z = a1()(state_b, wp, bp, m_b)
        uo, io = sc(u, l1, z.reshape(RB))
        uos.append(uo)
        ios.append(io)
    uo = jnp.concatenate(uos, axis=0)
    io = jnp.concatenate(ios, axis=0)
    return (io[:, :K], uo[:, :K])
